# Initial kernel scaffold; baseline (speedup 1.0000x reference)
import jax
import jax.numpy as jnp
from jax.experimental import pallas as pl


def kernel(input_ids, text_emb_w, speech_emb_w, text_pos_w, speech_pos_w):
    raise NotImplementedError("write your pallas kernel here")


# speedup vs baseline: 2.0765x; 2.0765x over previous
import functools

import jax
import jax.numpy as jnp
from jax import lax
from jax.experimental import pallas as pl
from jax.experimental.pallas import tpu as pltpu
from jax.experimental.pallas import tpu_sc as plsc

TEXT_VOCAB = 704
SPEECH_VOCAB = 8194
MAX_TEXT_SEQ = 2048 + 2
MAX_SPEECH_SEQ = 4096 + 4
HIDDEN = 1024
B, S = 4, 2048
N = B * S
NC, NS, L = 2, 16, 16
NW = NC * NS
CHUNK = N // NW
TPR = S // CHUNK
G = CHUNK // L
U = 32
PAD = CHUNK + U
UMAX = PAD // U


def _body(ids_hbm, te_hbm, se_hbm, tp_hbm, sp_hbm, out_hbm,
          ids_v, gi_t, pi_t, gi_s, pi_s, dt1, ds1, dt2, ds2,
          buf_a, buf_b, sem_g, sem_p, sem_s):
    wid = lax.axis_index("c") * NS + lax.axis_index("s")
    row = wid // TPR
    ls = (wid % TPR) * CHUNK
    base = wid * CHUNK
    iota = lax.iota(jnp.int32, L)

    pltpu.async_copy(ids_hbm.at[pl.ds(row * S, S)], ids_v, sem_g).wait()

    def pf_body(i, acc):
        v = ids_v[pl.ds(i * L, L)]
        return acc + jnp.sum((v < TEXT_VOCAB).astype(jnp.int32))
    tcarry0 = lax.fori_loop(0, ls // L, pf_body, jnp.int32(0))

    zvec = jnp.zeros((L,), jnp.int32)
    for q in range(PAD // L):
        gi_t[pl.ds(q * L, L)] = zvec
        pi_t[pl.ds(q * L, L)] = zvec
        gi_s[pl.ds(q * L, L)] = zvec
        pi_s[pl.ds(q * L, L)] = zvec
        dt1[pl.ds(q * L, L)] = zvec
        ds1[pl.ds(q * L, L)] = zvec

    def cp_body(g, carry):
        nt, ns, tc = carry
        v = ids_v[pl.ds(ls + g * L, L)]
        mt = v < TEXT_VOCAB
        inc_t = plsc.cumsum(mt.astype(jnp.int32))
        t = tc + inc_t
        s_in_row = ls + g * L + iota
        gidx = jnp.where(mt, v, v - TEXT_VOCAB)
        pidx = jnp.maximum(jnp.where(mt, t - 1, s_in_row - t), 0)
        dst = base + g * L + iota
        pos_t = nt + inc_t - 1
        plsc.store_scatter(gi_t, [pos_t], gidx, mask=mt)
        plsc.store_scatter(pi_t, [pos_t], pidx, mask=mt)
        plsc.store_scatter(dt1, [pos_t], dst, mask=mt)
        ms = jnp.logical_not(mt)
        pos_s = ns + plsc.cumsum(ms.astype(jnp.int32)) - 1
        plsc.store_scatter(gi_s, [pos_s], gidx, mask=ms)
        plsc.store_scatter(pi_s, [pos_s], pidx, mask=ms)
        plsc.store_scatter(ds1, [pos_s], dst, mask=ms)
        cnt = jnp.sum(mt.astype(jnp.int32))
        return nt + cnt, ns + (L - cnt), tc + cnt
    nt, ns, _ = lax.fori_loop(
        0, G, cp_body, (jnp.int32(0), jnp.int32(0), tcarry0))

    def pad_lists(n, gi, pi, d1):
        npad = ((n + U - 1) // U) * U
        zero16 = jnp.zeros((L,), jnp.int32)
        b_g = plsc.load_gather(gi, [zero16])
        b_p = plsc.load_gather(pi, [zero16])
        b_d = plsc.load_gather(d1, [zero16])
        a0 = (n // L) * L
        for k in range(2):
            off = a0 + k * L
            m = (off + iota) >= n
            gi[pl.ds(off, L)] = jnp.where(m, b_g, gi[pl.ds(off, L)])
            pi[pl.ds(off, L)] = jnp.where(m, b_p, pi[pl.ds(off, L)])
            d1[pl.ds(off, L)] = jnp.where(m, b_d, d1[pl.ds(off, L)])
        return npad

    ntp = pad_lists(nt, gi_t, pi_t, dt1)
    nsp = pad_lists(ns, gi_s, pi_s, ds1)

    for q in range(PAD // L):
        sl = pl.ds(q * L, L)
        gi_t[sl] = jnp.clip(gi_t[sl], 0, TEXT_VOCAB - 1)
        pi_t[sl] = jnp.clip(pi_t[sl], 0, MAX_TEXT_SEQ - 1)
        gi_s[sl] = jnp.clip(gi_s[sl], 0, SPEECH_VOCAB - 1)
        pi_s[sl] = jnp.clip(pi_s[sl], 0, MAX_SPEECH_SEQ - 1)

    for u in range(UMAX):
        for k in range(U // L):
            dt2[u, pl.ds(k * L, L)] = jnp.clip(
                dt1[pl.ds(u * U + k * L, L)], 0, N - 1)
            ds2[u, pl.ds(k * L, L)] = jnp.clip(
                ds1[pl.ds(u * U + k * L, L)], 0, N - 1)

    def run_units(nu, gi, pi, d2, tok_tab, pos_tab):
        def unit(u, _):
            ga = pltpu.async_copy(
                tok_tab.at[gi.at[pl.ds(u * U, U)]], buf_a, sem_g)
            gb = pltpu.async_copy(
                pos_tab.at[pi.at[pl.ds(u * U, U)]], buf_b, sem_p)
            ga.wait()
            gb.wait()

            def addrow(r, _):
                for k in range(HIDDEN // L):
                    plsc.addupdate(buf_a.at[r, pl.ds(k * L, L)],
                                   buf_b[r, pl.ds(k * L, L)])
                return 0
            lax.fori_loop(0, U, addrow, 0)
            pltpu.async_copy(buf_a, out_hbm.at[d2.at[u]], sem_s).wait()
            return 0
        lax.fori_loop(0, nu, unit, 0)

    run_units(ntp // U, gi_t, pi_t, dt2, te_hbm, tp_hbm)
    run_units(nsp // U, gi_s, pi_s, ds2, se_hbm, sp_hbm)


@functools.partial(
    pl.kernel,
    out_type=jax.ShapeDtypeStruct((N, HIDDEN), jnp.float32),
    mesh=plsc.VectorSubcoreMesh(core_axis_name="c", subcore_axis_name="s"),
    compiler_params=pltpu.CompilerParams(needs_layout_passes=False),
    scratch_types=[
        pltpu.VMEM((S,), jnp.int32),
        pltpu.VMEM((PAD,), jnp.int32),
        pltpu.VMEM((PAD,), jnp.int32),
        pltpu.VMEM((PAD,), jnp.int32),
        pltpu.VMEM((PAD,), jnp.int32),
        pltpu.VMEM((PAD,), jnp.int32),
        pltpu.VMEM((PAD,), jnp.int32),
        pltpu.VMEM((UMAX, U), jnp.int32),
        pltpu.VMEM((UMAX, U), jnp.int32),
        pltpu.VMEM((U, HIDDEN), jnp.float32),
        pltpu.VMEM((U, HIDDEN), jnp.float32),
        pltpu.SemaphoreType.DMA,
        pltpu.SemaphoreType.DMA,
        pltpu.SemaphoreType.DMA,
    ],
)
def _sc_embed(ids_hbm, te_hbm, se_hbm, tp_hbm, sp_hbm, out_hbm, *scratch):
    _body(ids_hbm, te_hbm, se_hbm, tp_hbm, sp_hbm, out_hbm, *scratch)


@jax.jit
def _run(input_ids, text_emb_w, speech_emb_w, text_pos_w, speech_pos_w):
    ids = input_ids.reshape(N).astype(jnp.int32)
    out = _sc_embed(ids, text_emb_w, speech_emb_w, text_pos_w, speech_pos_w)
    return out.reshape(B, S, HIDDEN)


def kernel(input_ids, text_emb_w, speech_emb_w, text_pos_w, speech_pos_w):
    return _run(input_ids, text_emb_w, speech_emb_w, text_pos_w, speech_pos_w)
